# baseline (speedup 1.0000x reference)
import functools
import jax
import jax.numpy as jnp
from jax import lax
from jax.experimental import pallas as pl
from jax.experimental.pallas import tpu as pltpu, tpu_sc as plsc

CHANNELS = 64
NVEC = CHANNELS // 16
EPS = 1e-5
NC, NS = 2, 16
NW = NC * NS
B, S = 4096, 200
NBT = B // 128
NUNITS = S * NBT
UPW = NUNITS // NW
OSTR = 129


def _rsqrt(x):
    i = lax.bitcast_convert_type(x, jnp.int32)
    i = jnp.int32(0x5F3759DF) - (i >> 1)
    y = lax.bitcast_convert_type(i, jnp.float32)
    for _ in range(3):
        y = y * (1.5 - 0.5 * x * y * y)
    return y


def _row_stats(vs):
    s = vs[0] + vs[1] + vs[2] + vs[3]
    q = vs[0] * vs[0] + vs[1] * vs[1] + vs[2] * vs[2] + vs[3] * vs[3]
    hs = jnp.sum(s)
    hq = jnp.sum(q)
    mu = hs * (1.0 / CHANNELS)
    var = hq * (1.0 / CHANNELS) - mu * mu
    return mu, _rsqrt(var + EPS)


def _pos_norm_kernel(pos_table, gamma, beta):
    ROWS = pos_table.shape[0]
    rpw = ROWS // NW

    @functools.partial(
        pl.kernel,
        out_type=jax.ShapeDtypeStruct((ROWS, CHANNELS), jnp.float32),
        mesh=plsc.VectorSubcoreMesh(core_axis_name="c", subcore_axis_name="s"),
        compiler_params=pltpu.CompilerParams(
            needs_layout_passes=False, use_tc_tiling_on_sc=False),
        scratch_types=[
            pltpu.VMEM((rpw, CHANNELS), jnp.float32),
            pltpu.VMEM((CHANNELS,), jnp.float32),
            pltpu.VMEM((CHANNELS,), jnp.float32),
        ],
    )
    def k(tab_hbm, g_hbm, b_hbm, out_hbm, buf, gv, bv):
        wid = lax.axis_index("s") * NC + lax.axis_index("c")
        base = wid * rpw
        pltpu.sync_copy(g_hbm, gv)
        pltpu.sync_copy(b_hbm, bv)
        pltpu.sync_copy(tab_hbm.at[pl.ds(base, rpw)], buf)
        g = [gv[pl.ds(16 * k, 16)] for k in range(NVEC)]
        b = [bv[pl.ds(16 * k, 16)] for k in range(NVEC)]

        def body(r, _):
            vs = [buf[r, pl.ds(16 * k, 16)] for k in range(NVEC)]
            mu, inv = _row_stats(vs)
            for k in range(NVEC):
                buf[r, pl.ds(16 * k, 16)] = (vs[k] - mu) * inv * g[k] + b[k]
            return 0

        lax.fori_loop(0, rpw, body, 0)
        pltpu.sync_copy(buf, out_hbm.at[pl.ds(base, rpw)])

    return k(pos_table, gamma, beta)


def _main_kernel(tok_idx, pos_idx, token_table, pos_norm, gamma, beta):

    @functools.partial(
        pl.kernel,
        out_type=jax.ShapeDtypeStruct((S, 8, NBT, 8, 128), jnp.float32),
        mesh=plsc.VectorSubcoreMesh(core_axis_name="c", subcore_axis_name="s"),
        compiler_params=pltpu.CompilerParams(
            needs_layout_passes=False, use_tc_tiling_on_sc=False),
        scratch_types=[
            pltpu.VMEM((UPW * 128,), jnp.int32),
            pltpu.VMEM((UPW * 128,), jnp.int32),
            pltpu.VMEM((2, 128, CHANNELS), jnp.float32),
            pltpu.VMEM((2, 128, CHANNELS), jnp.float32),
            pltpu.VMEM((2, CHANNELS, OSTR), jnp.float32),
            pltpu.VMEM((CHANNELS,), jnp.float32),
            pltpu.VMEM((CHANNELS,), jnp.float32),
            pltpu.SemaphoreType.DMA((2,)),
            pltpu.SemaphoreType.DMA((2,)),
            pltpu.SemaphoreType.DMA((2,)),
        ],
    )
    def k(ti_hbm, pi_hbm, tab_hbm, pn_hbm, g_hbm, b_hbm, out_hbm,
          idx_t, idx_p, tok_buf, pos_buf, obuf, gv, bv,
          sem_t, sem_p, sem_o):
        wid = lax.axis_index("s") * NC + lax.axis_index("c")
        ubase = wid * UPW
        pltpu.sync_copy(g_hbm, gv)
        pltpu.sync_copy(b_hbm, bv)
        pltpu.sync_copy(ti_hbm.at[pl.ds(ubase * 128, UPW * 128)], idx_t)
        pltpu.sync_copy(pi_hbm.at[pl.ds(ubase * 128, UPW * 128)], idx_p)
        g = [gv[pl.ds(16 * k, 16)] for k in range(NVEC)]
        b = [bv[pl.ds(16 * k, 16)] for k in range(NVEC)]

        iota = lax.broadcasted_iota(jnp.int32, (16,), 0)
        cvec = [16 * k + iota for k in range(NVEC)]

        def fire_gather(i, nb):
            pltpu.async_copy(
                tab_hbm.at[idx_t.at[pl.ds(i * 128, 128)]],
                tok_buf.at[nb], sem_t.at[nb])
            pltpu.async_copy(
                pn_hbm.at[idx_p.at[pl.ds(i * 128, 128)]],
                pos_buf.at[nb], sem_p.at[nb])

        def wait_gather(i, nb):
            pltpu.make_async_copy(
                tab_hbm.at[idx_t.at[pl.ds(i * 128, 128)]],
                tok_buf.at[nb], sem_t.at[nb]).wait()
            pltpu.make_async_copy(
                pn_hbm.at[idx_p.at[pl.ds(i * 128, 128)]],
                pos_buf.at[nb], sem_p.at[nb]).wait()

        def unit_su(i):
            u = ubase + i
            s = (u // (NBT * 8)) * 8 + u % 8
            bt = (u // 8) % NBT
            return s, bt

        def fire_out(i, nb):
            s, bt = unit_su(i)
            for ct in range(8):
                pltpu.async_copy(
                    obuf.at[nb, pl.ds(8 * ct, 8), pl.ds(0, 128)],
                    out_hbm.at[s, ct, bt], sem_o.at[nb])

        def wait_out(i, nb):
            s, bt = unit_su(i)
            for ct in range(8):
                pltpu.make_async_copy(
                    obuf.at[nb, pl.ds(8 * ct, 8), pl.ds(0, 128)],
                    out_hbm.at[s, ct, bt], sem_o.at[nb]).wait()

        def compute(nb):
            ob = obuf.at[nb]

            def row(r, _):
                vs = [tok_buf[nb, r, pl.ds(16 * k, 16)] for k in range(NVEC)]
                mu, inv = _row_stats(vs)
                rvec = jnp.full((16,), r, dtype=jnp.int32)
                for k in range(NVEC):
                    o = ((vs[k] - mu) * inv * g[k] + b[k]
                         + pos_buf[nb, r, pl.ds(16 * k, 16)])
                    plsc.store_scatter(ob, [cvec[k], rvec], o)
                return 0

            lax.fori_loop(0, 128, row, 0)

        fire_gather(0, 0)
        fire_gather(1, 1)
        for nb in (0, 1):
            wait_gather(nb, nb)
            compute(nb)
            fire_out(nb, nb)
            fire_gather(nb + 2, nb)

        def pair(i, _):
            for nb in (0, 1):
                gi = 2 * i + nb
                wait_gather(gi, nb)
                wait_out(gi - 2, nb)
                compute(nb)
                fire_out(gi, nb)
                fire_gather(gi + 2, nb)
            return 0

        lax.fori_loop(1, UPW // 2 - 1, pair, 0)

        for nb in (0, 1):
            gi = UPW - 2 + nb
            wait_gather(gi, nb)
            wait_out(gi - 2, nb)
            compute(nb)
            fire_out(gi, nb)
        for nb in (0, 1):
            wait_out(UPW - 2 + nb, nb)

    return k(tok_idx, pos_idx, token_table, pos_norm, gamma, beta)


@jax.jit
def kernel(token_x, time_step, token_table, pos_table,
           tok_gamma, tok_beta, pos_gamma, pos_beta):
    pos_norm = _pos_norm_kernel(pos_table, pos_gamma, pos_beta)

    def flat_native(ix):
        return (ix.T.reshape(S // 8, 8, NBT, 128)
                .transpose(0, 2, 1, 3).reshape(-1))

    out5 = _main_kernel(
        flat_native(token_x), flat_native(time_step),
        token_table, pos_norm, tok_gamma, tok_beta,
    )
    return out5.transpose(2, 4, 0, 1, 3).reshape(B, S, CHANNELS)


# speedup vs baseline: 1.6212x; 1.6212x over previous
import functools
import jax
import jax.numpy as jnp
from jax import lax
from jax.experimental import pallas as pl
from jax.experimental.pallas import tpu as pltpu, tpu_sc as plsc

CHANNELS = 64
NVEC = CHANNELS // 16
EPS = 1e-5


def _rsqrt(x):
    i = lax.bitcast_convert_type(x, jnp.int32)
    i = jnp.int32(0x5F3759DF) - (i >> 1)
    y = lax.bitcast_convert_type(i, jnp.float32)
    for _ in range(3):
        y = y * (1.5 - 0.5 * x * y * y)
    return y


def _row_stats(vs):
    s = vs[0] + vs[1] + vs[2] + vs[3]
    q = vs[0] * vs[0] + vs[1] * vs[1] + vs[2] * vs[2] + vs[3] * vs[3]
    hs = jnp.sum(s)
    hq = jnp.sum(q)
    mu = hs * (1.0 / CHANNELS)
    var = hq * (1.0 / CHANNELS) - mu * mu
    return mu, _rsqrt(var + EPS)


def _pos_norm_kernel(pos_table, gamma, beta):
    NC, NS = 2, 16
    NW = NC * NS
    ROWS = pos_table.shape[0]
    rpw = ROWS // NW

    @functools.partial(
        pl.kernel,
        out_type=jax.ShapeDtypeStruct((ROWS, CHANNELS), jnp.float32),
        mesh=plsc.VectorSubcoreMesh(core_axis_name="c", subcore_axis_name="s"),
        compiler_params=pltpu.CompilerParams(needs_layout_passes=False, use_tc_tiling_on_sc=False),
        scratch_types=[
            pltpu.VMEM((rpw, CHANNELS), jnp.float32),
            pltpu.VMEM((CHANNELS,), jnp.float32),
            pltpu.VMEM((CHANNELS,), jnp.float32),
        ],
    )
    def k(tab_hbm, g_hbm, b_hbm, out_hbm, buf, gv, bv):
        wid = lax.axis_index("s") * NC + lax.axis_index("c")
        base = wid * rpw
        pltpu.sync_copy(g_hbm, gv)
        pltpu.sync_copy(b_hbm, bv)
        pltpu.sync_copy(tab_hbm.at[pl.ds(base, rpw)], buf)
        g = [gv[pl.ds(16 * k, 16)] for k in range(NVEC)]
        b = [bv[pl.ds(16 * k, 16)] for k in range(NVEC)]

        def body(r, _):
            vs = [buf[r, pl.ds(16 * k, 16)] for k in range(NVEC)]
            mu, inv = _row_stats(vs)
            for k in range(NVEC):
                buf[r, pl.ds(16 * k, 16)] = (vs[k] - mu) * inv * g[k] + b[k]
            return 0

        lax.fori_loop(0, rpw, body, 0)
        pltpu.sync_copy(buf, out_hbm.at[pl.ds(base, rpw)])

    return k(pos_table, gamma, beta)


POS_ROWS = 2048


def _main_kernel(tok_idx, pos_idx, token_table, pos_table,
                 tok_gamma, tok_beta, pos_gamma, pos_beta):
    NC, NS = 2, 16
    NW = NC * NS
    PPW = POS_ROWS // NS
    N = tok_idx.shape[0]
    GROUP = 128
    rpw = N // NW
    ngroups = rpw // GROUP

    @functools.partial(
        pl.kernel,
        out_type=jax.ShapeDtypeStruct((N, CHANNELS), jnp.float32),
        mesh=plsc.VectorSubcoreMesh(core_axis_name="c", subcore_axis_name="s"),
        compiler_params=pltpu.CompilerParams(needs_layout_passes=False, use_tc_tiling_on_sc=False),
        scratch_types=[
            pltpu.VMEM((rpw,), jnp.int32),
            pltpu.VMEM((rpw,), jnp.int32),
            pltpu.VMEM((2, GROUP, CHANNELS), jnp.float32),
            pltpu.VMEM((2, GROUP, CHANNELS), jnp.float32),
            pltpu.VMEM((2, GROUP, CHANNELS), jnp.float32),
            pltpu.VMEM((PPW, CHANNELS), jnp.float32),
            pltpu.VMEM((CHANNELS,), jnp.float32),
            pltpu.VMEM((CHANNELS,), jnp.float32),
            pltpu.VMEM_SHARED((POS_ROWS, CHANNELS), jnp.float32),
            pltpu.SemaphoreType.DMA((2,)),
            pltpu.SemaphoreType.DMA((2,)),
            pltpu.SemaphoreType.DMA((2,)),
        ],
    )
    def k(ti_hbm, pi_hbm, tab_hbm, ptab_hbm, tg_hbm, tb_hbm, pg_hbm, pb_hbm,
          out_hbm, idx_t, idx_p, tok_buf, pos_buf, out_buf, pstage, gv, bv,
          pos_sp, sem_t, sem_p, sem_o):
        sid = lax.axis_index("s")
        wid = sid * NC + lax.axis_index("c")
        base = wid * rpw

        pltpu.sync_copy(pg_hbm, gv)
        pltpu.sync_copy(pb_hbm, bv)
        pltpu.sync_copy(ptab_hbm.at[pl.ds(sid * PPW, PPW)], pstage)
        pg = [gv[pl.ds(16 * k, 16)] for k in range(NVEC)]
        pb = [bv[pl.ds(16 * k, 16)] for k in range(NVEC)]

        def prow(r, _):
            vs = [pstage[r, pl.ds(16 * k, 16)] for k in range(NVEC)]
            mu, inv = _row_stats(vs)
            for k in range(NVEC):
                pstage[r, pl.ds(16 * k, 16)] = (vs[k] - mu) * inv * pg[k] + pb[k]
            return 0

        lax.fori_loop(0, PPW, prow, 0)
        pltpu.sync_copy(pstage, pos_sp.at[pl.ds(sid * PPW, PPW)])

        pltpu.sync_copy(ti_hbm.at[pl.ds(base, rpw)], idx_t)
        pltpu.sync_copy(pi_hbm.at[pl.ds(base, rpw)], idx_p)
        pltpu.sync_copy(tg_hbm, gv)
        pltpu.sync_copy(tb_hbm, bv)
        g = [gv[pl.ds(16 * k, 16)] for k in range(NVEC)]
        b = [bv[pl.ds(16 * k, 16)] for k in range(NVEC)]
        plsc.subcore_barrier()

        def fire_gather(gi, nb):
            pltpu.async_copy(
                tab_hbm.at[idx_t.at[pl.ds(gi * GROUP, GROUP)]],
                tok_buf.at[nb], sem_t.at[nb])
            pltpu.async_copy(
                pos_sp.at[idx_p.at[pl.ds(gi * GROUP, GROUP)]],
                pos_buf.at[nb], sem_p.at[nb])

        def wait_gather(gi, nb):
            pltpu.make_async_copy(tab_hbm.at[idx_t.at[pl.ds(gi * GROUP, GROUP)]],
                                  tok_buf.at[nb], sem_t.at[nb]).wait()
            pltpu.make_async_copy(pos_sp.at[idx_p.at[pl.ds(gi * GROUP, GROUP)]],
                                  pos_buf.at[nb], sem_p.at[nb]).wait()

        def fire_out(gi, nb):
            pltpu.async_copy(out_buf.at[nb],
                             out_hbm.at[pl.ds(base + gi * GROUP, GROUP)],
                             sem_o.at[nb])

        def wait_out(gi, nb):
            pltpu.make_async_copy(out_buf.at[nb],
                                  out_hbm.at[pl.ds(base + gi * GROUP, GROUP)],
                                  sem_o.at[nb]).wait()

        def compute(nb):
            def row(r, _):
                vs = [tok_buf[nb, r, pl.ds(16 * k, 16)] for k in range(NVEC)]
                mu, inv = _row_stats(vs)
                for k in range(NVEC):
                    out_buf[nb, r, pl.ds(16 * k, 16)] = (
                        (vs[k] - mu) * inv * g[k] + b[k]
                        + pos_buf[nb, r, pl.ds(16 * k, 16)]
                    )
                return 0

            lax.fori_loop(0, GROUP, row, 0)

        fire_gather(0, 0)
        fire_gather(1, 1)
        for nb in (0, 1):
            wait_gather(nb, nb)
            compute(nb)
            fire_out(nb, nb)
            fire_gather(nb + 2, nb)

        def pair(i, _):
            for nb in (0, 1):
                gi = 2 * i + nb
                wait_gather(gi, nb)
                wait_out(gi - 2, nb)
                compute(nb)
                fire_out(gi, nb)
                fire_gather(gi + 2, nb)
            return 0

        lax.fori_loop(1, ngroups // 2 - 1, pair, 0)

        for nb in (0, 1):
            gi = ngroups - 2 + nb
            wait_gather(gi, nb)
            wait_out(gi - 2, nb)
            compute(nb)
            fire_out(gi, nb)
        for nb in (0, 1):
            wait_out(ngroups - 2 + nb, nb)

    return k(tok_idx, pos_idx, token_table, pos_table,
             tok_gamma, tok_beta, pos_gamma, pos_beta)


@jax.jit
def kernel(token_x, time_step, token_table, pos_table,
           tok_gamma, tok_beta, pos_gamma, pos_beta):
    B, S = token_x.shape
    out = _main_kernel(
        token_x.reshape(-1), time_step.reshape(-1),
        token_table, pos_table, tok_gamma, tok_beta, pos_gamma, pos_beta,
    )
    return out.reshape(B, S, CHANNELS)


# speedup vs baseline: 1.6248x; 1.0022x over previous
import functools
import jax
import jax.numpy as jnp
from jax import lax
from jax.experimental import pallas as pl
from jax.experimental.pallas import tpu as pltpu, tpu_sc as plsc

CHANNELS = 64
NVEC = CHANNELS // 16
EPS = 1e-5


def _rsqrt(x):
    i = lax.bitcast_convert_type(x, jnp.int32)
    i = jnp.int32(0x5F3759DF) - (i >> 1)
    y = lax.bitcast_convert_type(i, jnp.float32)
    for _ in range(3):
        y = y * (1.5 - 0.5 * x * y * y)
    return y


def _row_stats(vs):
    s = vs[0] + vs[1] + vs[2] + vs[3]
    q = vs[0] * vs[0] + vs[1] * vs[1] + vs[2] * vs[2] + vs[3] * vs[3]
    hs = jnp.sum(s)
    hq = jnp.sum(q)
    mu = hs * (1.0 / CHANNELS)
    var = hq * (1.0 / CHANNELS) - mu * mu
    return mu, _rsqrt(var + EPS)


POS_ROWS = 2048


def _main_kernel(tok_idx, pos_idx, token_table, pos_table,
                 tok_gamma, tok_beta, pos_gamma, pos_beta):
    NC, NS = 2, 16
    NW = NC * NS
    PPW = POS_ROWS // NS
    N = tok_idx.shape[0]
    GROUP = 128
    rpw = N // NW
    ngroups = rpw // GROUP

    @functools.partial(
        pl.kernel,
        out_type=jax.ShapeDtypeStruct((N, CHANNELS), jnp.float32),
        mesh=plsc.VectorSubcoreMesh(core_axis_name="c", subcore_axis_name="s"),
        compiler_params=pltpu.CompilerParams(needs_layout_passes=False, use_tc_tiling_on_sc=False),
        scratch_types=[
            pltpu.VMEM((rpw,), jnp.int32),
            pltpu.VMEM((rpw,), jnp.int32),
            pltpu.VMEM((2, GROUP, CHANNELS), jnp.float32),
            pltpu.VMEM((2, GROUP, CHANNELS), jnp.float32),
            pltpu.VMEM((2, GROUP, CHANNELS), jnp.float32),
            pltpu.VMEM((PPW, CHANNELS), jnp.float32),
            pltpu.VMEM((CHANNELS,), jnp.float32),
            pltpu.VMEM((CHANNELS,), jnp.float32),
            pltpu.VMEM_SHARED((POS_ROWS, CHANNELS), jnp.float32),
            pltpu.SemaphoreType.DMA((2,)),
            pltpu.SemaphoreType.DMA((2,)),
            pltpu.SemaphoreType.DMA((2,)),
        ],
    )
    def k(ti_hbm, pi_hbm, tab_hbm, ptab_hbm, tg_hbm, tb_hbm, pg_hbm, pb_hbm,
          out_hbm, idx_t, idx_p, tok_buf, pos_buf, out_buf, pstage, gv, bv,
          pos_sp, sem_t, sem_p, sem_o):
        sid = lax.axis_index("s")
        wid = sid * NC + lax.axis_index("c")
        base = wid * rpw

        pltpu.sync_copy(pg_hbm, gv)
        pltpu.sync_copy(pb_hbm, bv)
        pltpu.sync_copy(ptab_hbm.at[pl.ds(sid * PPW, PPW)], pstage)
        pg = [gv[pl.ds(16 * k, 16)] for k in range(NVEC)]
        pb = [bv[pl.ds(16 * k, 16)] for k in range(NVEC)]

        def prow(r, _):
            vs = [pstage[r, pl.ds(16 * k, 16)] for k in range(NVEC)]
            mu, inv = _row_stats(vs)
            for k in range(NVEC):
                pstage[r, pl.ds(16 * k, 16)] = (vs[k] - mu) * inv * pg[k] + pb[k]
            return 0

        lax.fori_loop(0, PPW, prow, 0)
        pltpu.sync_copy(pstage, pos_sp.at[pl.ds(sid * PPW, PPW)])

        pltpu.sync_copy(ti_hbm.at[pl.ds(base, rpw)], idx_t)
        pltpu.sync_copy(pi_hbm.at[pl.ds(base, rpw)], idx_p)
        pltpu.sync_copy(tg_hbm, gv)
        pltpu.sync_copy(tb_hbm, bv)
        g = [gv[pl.ds(16 * k, 16)] for k in range(NVEC)]
        b = [bv[pl.ds(16 * k, 16)] for k in range(NVEC)]
        plsc.subcore_barrier()

        def fire_gather(gi, nb):
            pltpu.async_copy(
                tab_hbm.at[idx_t.at[pl.ds(gi * GROUP, GROUP)]],
                tok_buf.at[nb], sem_t.at[nb])
            pltpu.async_copy(
                pos_sp.at[idx_p.at[pl.ds(gi * GROUP, GROUP)]],
                pos_buf.at[nb], sem_p.at[nb])

        def wait_gather(gi, nb):
            pltpu.make_async_copy(tab_hbm.at[idx_t.at[pl.ds(gi * GROUP, GROUP)]],
                                  tok_buf.at[nb], sem_t.at[nb]).wait()
            pltpu.make_async_copy(pos_sp.at[idx_p.at[pl.ds(gi * GROUP, GROUP)]],
                                  pos_buf.at[nb], sem_p.at[nb]).wait()

        def fire_out(gi, nb):
            pltpu.async_copy(out_buf.at[nb],
                             out_hbm.at[pl.ds(base + gi * GROUP, GROUP)],
                             sem_o.at[nb])

        def wait_out(gi, nb):
            pltpu.make_async_copy(out_buf.at[nb],
                                  out_hbm.at[pl.ds(base + gi * GROUP, GROUP)],
                                  sem_o.at[nb]).wait()

        def compute(nb):
            def row(r, _):
                vs = [tok_buf[nb, r, pl.ds(16 * k, 16)] for k in range(NVEC)]
                mu, inv = _row_stats(vs)
                for k in range(NVEC):
                    out_buf[nb, r, pl.ds(16 * k, 16)] = (
                        (vs[k] - mu) * inv * g[k] + b[k]
                        + pos_buf[nb, r, pl.ds(16 * k, 16)]
                    )
                return 0

            lax.fori_loop(0, GROUP, row, 0)

        fire_gather(0, 0)
        fire_gather(1, 1)
        for nb in (0, 1):
            wait_gather(nb, nb)
            compute(nb)
            fire_out(nb, nb)
            fire_gather(nb + 2, nb)

        def pair(i, _):
            for nb in (0, 1):
                gi = 2 * i + nb
                wait_gather(gi, nb)
                wait_out(gi - 2, nb)
                compute(nb)
                fire_out(gi, nb)
                fire_gather(gi + 2, nb)
            return 0

        lax.fori_loop(1, ngroups // 2 - 1, pair, 0)

        for nb in (0, 1):
            gi = ngroups - 2 + nb
            wait_gather(gi, nb)
            wait_out(gi - 2, nb)
            compute(nb)
            fire_out(gi, nb)
        for nb in (0, 1):
            wait_out(ngroups - 2 + nb, nb)

    return k(tok_idx, pos_idx, token_table, pos_table,
             tok_gamma, tok_beta, pos_gamma, pos_beta)


@jax.jit
def kernel(token_x, time_step, token_table, pos_table,
           tok_gamma, tok_beta, pos_gamma, pos_beta):
    B, S = token_x.shape
    out = _main_kernel(
        token_x.reshape(-1), time_step.reshape(-1),
        token_table, pos_table, tok_gamma, tok_beta, pos_gamma, pos_beta,
    )
    return out.reshape(B, S, CHANNELS)
